# baseline (speedup 1.0000x reference)
import functools

import jax
import jax.numpy as jnp
from jax import lax
from jax.experimental import pallas as pl
from jax.experimental.pallas import tpu as pltpu
from jax.experimental.pallas import tpu_sc as plsc

_NC = 2
_NS = 16
_NW = _NC * _NS
_SUB = 128
_PACK = 4
_QROWS = 262144
_ROWS_PAD = _PACK * _QROWS


def _make_tc_pack_table(N, D):
    blk = 1024
    nblk = _QROWS // blk
    grid = (nblk,)

    def body(t0_ref, t1_ref, t2_ref, t3_ref, out_ref):
        eye = jnp.eye(D, dtype=jnp.float32)
        for j, t_ref in enumerate((t0_ref, t1_ref, t2_ref, t3_ref)):
            out_ref[:, pl.ds(j * D, D)] = lax.dot_general(
                t_ref[...], eye, (((0,), (0,)), ((), ())),
                preferred_element_type=jnp.float32)

    max_blk = (N + blk - 1) // blk - 1

    def in_spec(j):
        return pl.BlockSpec(
            (D, blk), lambda i, j=j: (0, jnp.minimum(nblk * j + i, max_blk)))

    return pl.pallas_call(
        body,
        grid=grid,
        in_specs=[in_spec(j) for j in range(_PACK)],
        out_specs=pl.BlockSpec((blk, _PACK * D), lambda i: (i, 0)),
        out_shape=jax.ShapeDtypeStruct((_QROWS, _PACK * D), jnp.float32),
    )


def _make_sc_pooled_gather(B, L, D):
    b_per_w = B // _NW
    nsub = b_per_w // _SUB
    mesh = plsc.VectorSubcoreMesh(
        core_axis_name="c", subcore_axis_name="s",
        num_cores=_NC, num_subcores=_NS)

    @functools.partial(
        pl.kernel,
        out_type=(jax.ShapeDtypeStruct((B, D), jnp.float32),
                  jax.ShapeDtypeStruct((B, D), jnp.float32)),
        mesh=mesh,
        scratch_types=[
            pltpu.VMEM((L + 1, nsub, _SUB), jnp.int32),
            pltpu.VMEM((L + 1, nsub, _SUB), jnp.int32),
            pltpu.VMEM((b_per_w, D), jnp.float32),
            pltpu.VMEM((b_per_w, D), jnp.float32),
            pltpu.SemaphoreType.DMA,
        ],
        compiler_params=pltpu.CompilerParams(use_tc_tiling_on_sc=False,
                                             needs_layout_passes=False),
    )
    def sc_kernel(enc1t, enc2t, emb_hbm, e1_out, e2_out,
                  idx1_v, idx2_v, acc1_v, acc2_v, sem):
        wid = lax.axis_index("s") * _NC + lax.axis_index("c")
        base = wid * b_per_w

        pltpu.sync_copy(enc1t.at[:, pl.ds(wid * nsub, nsub), :],
                        idx1_v.at[pl.ds(0, L)])
        pltpu.sync_copy(enc2t.at[:, pl.ds(wid * nsub, nsub), :],
                        idx2_v.at[pl.ds(0, L)])

        def remap(l):
            for idx_v in (idx1_v, idx2_v):
                for c in range(nsub):
                    for g in range(_SUB // 16):
                        v = idx_v[l, c, pl.ds(g * 16, 16)]
                        idx_v[l, c, pl.ds(g * 16, 16)] = (
                            (v & (_QROWS - 1)) * _PACK + (v >> 18))

        def fire(l, add):
            ds = []
            for idx_v, acc_v in ((idx1_v, acc1_v), (idx2_v, acc2_v)):
                for c in range(nsub):
                    ds.append(pltpu.async_copy(
                        emb_hbm.at[idx_v.at[l, c]],
                        acc_v.at[pl.ds(c * _SUB, _SUB)],
                        sem, add=add))
            return ds

        def remap_body(l, carry):
            remap(l)
            return carry

        lax.fori_loop(0, L, remap_body, 0)

        for d in fire(0, False):
            d.wait()

        def body(l, carry):
            for d in fire(l, True):
                d.wait()
            return carry

        lax.fori_loop(1, L, body, 0)

        pltpu.sync_copy(acc1_v, e1_out.at[pl.ds(base, b_per_w)])
        pltpu.sync_copy(acc2_v, e2_out.at[pl.ds(base, b_per_w)])

    return sc_kernel


def _make_tc_mlp(B, L, D, n_types, nhid, nclasses):
    blk = 512
    grid = (B // blk,)

    def body(e1_ref, e2_ref, c1_ref, c2_ref, w1_ref, b1_ref, w2_ref, b2_ref,
             out_ref):
        n1 = jnp.sum((c1_ref[...] != n_types).astype(jnp.float32), axis=0,
                     keepdims=True).T
        n2 = jnp.sum((c2_ref[...] != n_types).astype(jnp.float32), axis=0,
                     keepdims=True).T
        e1 = e1_ref[...] / n1
        e2 = e2_ref[...] / n2
        feat = jnp.concatenate([e1, e2, e1 * e2, jnp.abs(e1 - e2)], axis=1)
        h = lax.dot_general(feat, w1_ref[...], (((1,), (1,)), ((), ())),
                            preferred_element_type=jnp.float32) + b1_ref[...]
        h = jax.nn.sigmoid(h)
        out_ref[...] = lax.dot_general(h, w2_ref[...], (((1,), (1,)), ((), ())),
                                       preferred_element_type=jnp.float32
                                       ) + b2_ref[...]

    return pl.pallas_call(
        body,
        grid=grid,
        in_specs=[
            pl.BlockSpec((blk, D), lambda i: (i, 0)),
            pl.BlockSpec((blk, D), lambda i: (i, 0)),
            pl.BlockSpec((L, blk), lambda i: (0, i)),
            pl.BlockSpec((L, blk), lambda i: (0, i)),
            pl.BlockSpec((nhid, 4 * D), lambda i: (0, 0)),
            pl.BlockSpec((1, nhid), lambda i: (0, 0)),
            pl.BlockSpec((nclasses, nhid), lambda i: (0, 0)),
            pl.BlockSpec((1, nclasses), lambda i: (0, 0)),
        ],
        out_specs=pl.BlockSpec((blk, nclasses), lambda i: (i, 0)),
        out_shape=jax.ShapeDtypeStruct((B, nclasses), jnp.float32),
    )


def kernel(enc1, enc2, emb, W1, b1, W2, b2):
    B, L = enc1.shape
    N, D = emb.shape
    n_types = N - 1
    nhid = W1.shape[0]
    nclasses = W2.shape[0]

    enc1t = enc1.T.reshape(L, B // _SUB, _SUB)
    enc2t = enc2.T.reshape(L, B // _SUB, _SUB)
    embT = emb.T
    emb_flat = _make_tc_pack_table(N, D)(embT, embT, embT, embT)
    emb_sc = emb_flat.reshape(_ROWS_PAD, D)

    e1_sum, e2_sum = _make_sc_pooled_gather(B, L, D)(enc1t, enc2t, emb_sc)
    return _make_tc_mlp(B, L, D, n_types, nhid, nclasses)(
        e1_sum, e2_sum, enc1.T, enc2.T, W1, b1.reshape(1, nhid),
        W2, b2.reshape(1, nclasses))


# speedup vs baseline: 1.1134x; 1.1134x over previous
import functools

import jax
import jax.numpy as jnp
from jax import lax
from jax.experimental import pallas as pl
from jax.experimental.pallas import tpu as pltpu
from jax.experimental.pallas import tpu_sc as plsc

_NC = 2
_NS = 16
_NW = _NC * _NS
_SUB = 128
_PACK = 4
_QROWS = 262144
_ROWS_PAD = _PACK * _QROWS


def _make_tc_pack_table(N, D):
    blk = 4096
    nblk = _QROWS // blk
    grid = (nblk,)

    def body(t0_ref, t1_ref, t2_ref, t3_ref, out_ref):
        eye = jnp.eye(D, dtype=jnp.float32)
        for j, t_ref in enumerate((t0_ref, t1_ref, t2_ref, t3_ref)):
            out_ref[:, pl.ds(j * D, D)] = lax.dot_general(
                t_ref[...], eye, (((0,), (0,)), ((), ())),
                preferred_element_type=jnp.float32)

    max_blk = (N + blk - 1) // blk - 1

    def in_spec(j):
        return pl.BlockSpec(
            (D, blk), lambda i, j=j: (0, jnp.minimum(nblk * j + i, max_blk)))

    return pl.pallas_call(
        body,
        grid=grid,
        in_specs=[in_spec(j) for j in range(_PACK)],
        out_specs=pl.BlockSpec((blk, _PACK * D), lambda i: (i, 0)),
        out_shape=jax.ShapeDtypeStruct((_QROWS, _PACK * D), jnp.float32),
    )


def _make_sc_pooled_gather(B, L, D):
    b_per_w = B // _NW
    nsub = b_per_w // _SUB
    mesh = plsc.VectorSubcoreMesh(
        core_axis_name="c", subcore_axis_name="s",
        num_cores=_NC, num_subcores=_NS)

    @functools.partial(
        pl.kernel,
        out_type=(jax.ShapeDtypeStruct((B, D), jnp.float32),
                  jax.ShapeDtypeStruct((B, D), jnp.float32)),
        mesh=mesh,
        scratch_types=[
            pltpu.VMEM((L + 1, nsub, _SUB), jnp.int32),
            pltpu.VMEM((L + 1, nsub, _SUB), jnp.int32),
            pltpu.VMEM((b_per_w, D), jnp.float32),
            pltpu.VMEM((b_per_w, D), jnp.float32),
            pltpu.SemaphoreType.DMA,
        ],
        compiler_params=pltpu.CompilerParams(use_tc_tiling_on_sc=False,
                                             needs_layout_passes=False),
    )
    def sc_kernel(enc1t, enc2t, emb_hbm, e1_out, e2_out,
                  idx1_v, idx2_v, acc1_v, acc2_v, sem):
        wid = lax.axis_index("s") * _NC + lax.axis_index("c")
        base = wid * b_per_w

        pltpu.sync_copy(enc1t.at[:, pl.ds(wid * nsub, nsub), :],
                        idx1_v.at[pl.ds(0, L)])
        pltpu.sync_copy(enc2t.at[:, pl.ds(wid * nsub, nsub), :],
                        idx2_v.at[pl.ds(0, L)])

        def remap(l):
            for idx_v in (idx1_v, idx2_v):
                for c in range(nsub):
                    for g in range(_SUB // 16):
                        v = idx_v[l, c, pl.ds(g * 16, 16)]
                        idx_v[l, c, pl.ds(g * 16, 16)] = (
                            (v & (_QROWS - 1)) * _PACK + (v >> 18))

        def fire(l, add):
            ds = []
            for idx_v, acc_v in ((idx1_v, acc1_v), (idx2_v, acc2_v)):
                for c in range(nsub):
                    ds.append(pltpu.async_copy(
                        emb_hbm.at[idx_v.at[l, c]],
                        acc_v.at[pl.ds(c * _SUB, _SUB)],
                        sem, add=add))
            return ds

        def remap_body(l, carry):
            remap(l)
            return carry

        lax.fori_loop(0, L, remap_body, 0)

        for d in fire(0, False):
            d.wait()

        def body(l, carry):
            for d in fire(l, True):
                d.wait()
            return carry

        lax.fori_loop(1, L, body, 0)

        pltpu.sync_copy(acc1_v, e1_out.at[pl.ds(base, b_per_w)])
        pltpu.sync_copy(acc2_v, e2_out.at[pl.ds(base, b_per_w)])

    return sc_kernel


def _make_tc_mlp(B, L, D, n_types, nhid, nclasses):
    blk = 512
    grid = (B // blk,)

    def body(e1_ref, e2_ref, c1_ref, c2_ref, w1_ref, b1_ref, w2_ref, b2_ref,
             out_ref):
        n1 = jnp.sum((c1_ref[...] != n_types).astype(jnp.float32), axis=0,
                     keepdims=True).T
        n2 = jnp.sum((c2_ref[...] != n_types).astype(jnp.float32), axis=0,
                     keepdims=True).T
        e1 = e1_ref[...] / n1
        e2 = e2_ref[...] / n2
        feat = jnp.concatenate([e1, e2, e1 * e2, jnp.abs(e1 - e2)], axis=1)
        h = lax.dot_general(feat, w1_ref[...], (((1,), (1,)), ((), ())),
                            preferred_element_type=jnp.float32) + b1_ref[...]
        h = jax.nn.sigmoid(h)
        out_ref[...] = lax.dot_general(h, w2_ref[...], (((1,), (1,)), ((), ())),
                                       preferred_element_type=jnp.float32
                                       ) + b2_ref[...]

    return pl.pallas_call(
        body,
        grid=grid,
        in_specs=[
            pl.BlockSpec((blk, D), lambda i: (i, 0)),
            pl.BlockSpec((blk, D), lambda i: (i, 0)),
            pl.BlockSpec((L, blk), lambda i: (0, i)),
            pl.BlockSpec((L, blk), lambda i: (0, i)),
            pl.BlockSpec((nhid, 4 * D), lambda i: (0, 0)),
            pl.BlockSpec((1, nhid), lambda i: (0, 0)),
            pl.BlockSpec((nclasses, nhid), lambda i: (0, 0)),
            pl.BlockSpec((1, nclasses), lambda i: (0, 0)),
        ],
        out_specs=pl.BlockSpec((blk, nclasses), lambda i: (i, 0)),
        out_shape=jax.ShapeDtypeStruct((B, nclasses), jnp.float32),
    )


def kernel(enc1, enc2, emb, W1, b1, W2, b2):
    B, L = enc1.shape
    N, D = emb.shape
    n_types = N - 1
    nhid = W1.shape[0]
    nclasses = W2.shape[0]

    enc1t = enc1.T.reshape(L, B // _SUB, _SUB)
    enc2t = enc2.T.reshape(L, B // _SUB, _SUB)
    embT = emb.T
    emb_flat = _make_tc_pack_table(N, D)(embT, embT, embT, embT)
    emb_sc = emb_flat.reshape(_ROWS_PAD, D)

    e1_sum, e2_sum = _make_sc_pooled_gather(B, L, D)(enc1t, enc2t, emb_sc)
    return _make_tc_mlp(B, L, D, n_types, nhid, nclasses)(
        e1_sum, e2_sum, enc1.T, enc2.T, W1, b1.reshape(1, nhid),
        W2, b2.reshape(1, nclasses))


# speedup vs baseline: 1.1245x; 1.0099x over previous
import functools

import jax
import jax.numpy as jnp
from jax import lax
from jax.experimental import pallas as pl
from jax.experimental.pallas import tpu as pltpu
from jax.experimental.pallas import tpu_sc as plsc

_NC = 2
_NS = 16
_NW = _NC * _NS
_SUB = 128
_PACK = 4
_QROWS = 262144
_ROWS_PAD = _PACK * _QROWS


def _make_tc_pack_table(N, D):
    blk = 8192
    nblk = _QROWS // blk
    grid = (nblk,)

    def body(t0_ref, t1_ref, t2_ref, t3_ref, out_ref):
        eye = jnp.eye(D, dtype=jnp.float32)
        for j, t_ref in enumerate((t0_ref, t1_ref, t2_ref, t3_ref)):
            out_ref[:, pl.ds(j * D, D)] = lax.dot_general(
                t_ref[...], eye, (((0,), (0,)), ((), ())),
                preferred_element_type=jnp.float32)

    max_blk = (N + blk - 1) // blk - 1

    def in_spec(j):
        return pl.BlockSpec(
            (D, blk), lambda i, j=j: (0, jnp.minimum(nblk * j + i, max_blk)))

    return pl.pallas_call(
        body,
        grid=grid,
        in_specs=[in_spec(j) for j in range(_PACK)],
        out_specs=pl.BlockSpec((blk, _PACK * D), lambda i: (i, 0)),
        out_shape=jax.ShapeDtypeStruct((_QROWS, _PACK * D), jnp.float32),
    )


def _make_sc_pooled_gather(B, L, D):
    b_per_w = B // _NW
    nsub = b_per_w // _SUB
    mesh = plsc.VectorSubcoreMesh(
        core_axis_name="c", subcore_axis_name="s",
        num_cores=_NC, num_subcores=_NS)

    @functools.partial(
        pl.kernel,
        out_type=(jax.ShapeDtypeStruct((B, D), jnp.float32),
                  jax.ShapeDtypeStruct((B, D), jnp.float32)),
        mesh=mesh,
        scratch_types=[
            pltpu.VMEM((L + 1, nsub, _SUB), jnp.int32),
            pltpu.VMEM((L + 1, nsub, _SUB), jnp.int32),
            pltpu.VMEM((b_per_w, D), jnp.float32),
            pltpu.VMEM((b_per_w, D), jnp.float32),
            pltpu.SemaphoreType.DMA,
        ],
        compiler_params=pltpu.CompilerParams(use_tc_tiling_on_sc=False,
                                             needs_layout_passes=False),
    )
    def sc_kernel(enc1t, enc2t, emb_hbm, e1_out, e2_out,
                  idx1_v, idx2_v, acc1_v, acc2_v, sem):
        wid = lax.axis_index("s") * _NC + lax.axis_index("c")
        base = wid * b_per_w

        pltpu.sync_copy(enc1t.at[:, pl.ds(wid * nsub, nsub), :],
                        idx1_v.at[pl.ds(0, L)])
        pltpu.sync_copy(enc2t.at[:, pl.ds(wid * nsub, nsub), :],
                        idx2_v.at[pl.ds(0, L)])

        def remap(l):
            for idx_v in (idx1_v, idx2_v):
                for c in range(nsub):
                    for g in range(_SUB // 16):
                        v = idx_v[l, c, pl.ds(g * 16, 16)]
                        idx_v[l, c, pl.ds(g * 16, 16)] = (
                            (v & (_QROWS - 1)) * _PACK + (v >> 18))

        def fire(l, add):
            ds = []
            for idx_v, acc_v in ((idx1_v, acc1_v), (idx2_v, acc2_v)):
                for c in range(nsub):
                    ds.append(pltpu.async_copy(
                        emb_hbm.at[idx_v.at[l, c]],
                        acc_v.at[pl.ds(c * _SUB, _SUB)],
                        sem, add=add))
            return ds

        def remap_body(l, carry):
            remap(l)
            return carry

        lax.fori_loop(0, L, remap_body, 0)

        for d in fire(0, False):
            d.wait()

        def body(l, carry):
            for d in fire(l, True):
                d.wait()
            return carry

        lax.fori_loop(1, L, body, 0)

        pltpu.sync_copy(acc1_v, e1_out.at[pl.ds(base, b_per_w)])
        pltpu.sync_copy(acc2_v, e2_out.at[pl.ds(base, b_per_w)])

    return sc_kernel


def _make_tc_mlp(B, L, D, n_types, nhid, nclasses):
    blk = 512
    grid = (B // blk,)

    def body(e1_ref, e2_ref, c1_ref, c2_ref, w1_ref, b1_ref, w2_ref, b2_ref,
             out_ref):
        n1 = jnp.sum((c1_ref[...] != n_types).astype(jnp.float32), axis=0,
                     keepdims=True).T
        n2 = jnp.sum((c2_ref[...] != n_types).astype(jnp.float32), axis=0,
                     keepdims=True).T
        e1 = e1_ref[...] / n1
        e2 = e2_ref[...] / n2
        feat = jnp.concatenate([e1, e2, e1 * e2, jnp.abs(e1 - e2)], axis=1)
        h = lax.dot_general(feat, w1_ref[...], (((1,), (1,)), ((), ())),
                            preferred_element_type=jnp.float32) + b1_ref[...]
        h = jax.nn.sigmoid(h)
        out_ref[...] = lax.dot_general(h, w2_ref[...], (((1,), (1,)), ((), ())),
                                       preferred_element_type=jnp.float32
                                       ) + b2_ref[...]

    return pl.pallas_call(
        body,
        grid=grid,
        in_specs=[
            pl.BlockSpec((blk, D), lambda i: (i, 0)),
            pl.BlockSpec((blk, D), lambda i: (i, 0)),
            pl.BlockSpec((L, blk), lambda i: (0, i)),
            pl.BlockSpec((L, blk), lambda i: (0, i)),
            pl.BlockSpec((nhid, 4 * D), lambda i: (0, 0)),
            pl.BlockSpec((1, nhid), lambda i: (0, 0)),
            pl.BlockSpec((nclasses, nhid), lambda i: (0, 0)),
            pl.BlockSpec((1, nclasses), lambda i: (0, 0)),
        ],
        out_specs=pl.BlockSpec((blk, nclasses), lambda i: (i, 0)),
        out_shape=jax.ShapeDtypeStruct((B, nclasses), jnp.float32),
    )


def kernel(enc1, enc2, emb, W1, b1, W2, b2):
    B, L = enc1.shape
    N, D = emb.shape
    n_types = N - 1
    nhid = W1.shape[0]
    nclasses = W2.shape[0]

    enc1t = enc1.T.reshape(L, B // _SUB, _SUB)
    enc2t = enc2.T.reshape(L, B // _SUB, _SUB)
    embT = emb.T
    emb_flat = _make_tc_pack_table(N, D)(embT, embT, embT, embT)
    emb_sc = emb_flat.reshape(_ROWS_PAD, D)

    e1_sum, e2_sum = _make_sc_pooled_gather(B, L, D)(enc1t, enc2t, emb_sc)
    return _make_tc_mlp(B, L, D, n_types, nhid, nclasses)(
        e1_sum, e2_sum, enc1.T, enc2.T, W1, b1.reshape(1, nhid),
        W2, b2.reshape(1, nclasses))


# speedup vs baseline: 1.1247x; 1.0002x over previous
import functools

import jax
import jax.numpy as jnp
from jax import lax
from jax.experimental import pallas as pl
from jax.experimental.pallas import tpu as pltpu
from jax.experimental.pallas import tpu_sc as plsc

_NC = 2
_NS = 16
_NW = _NC * _NS
_SUB = 128
_PACK = 4
_QROWS = 262144
_ROWS_PAD = _PACK * _QROWS


def _make_tc_pack_table(N, D):
    blk = 8192
    nblk = _QROWS // blk
    grid = (nblk,)

    def body(t0_ref, t1_ref, t2_ref, t3_ref, out_ref):
        eye = jnp.eye(D, dtype=jnp.float32)
        for j, t_ref in enumerate((t0_ref, t1_ref, t2_ref, t3_ref)):
            out_ref[:, pl.ds(j * D, D)] = lax.dot_general(
                t_ref[...], eye, (((0,), (0,)), ((), ())),
                preferred_element_type=jnp.float32)

    max_blk = (N + blk - 1) // blk - 1

    def in_spec(j):
        return pl.BlockSpec(
            (D, blk), lambda i, j=j: (0, jnp.minimum(nblk * j + i, max_blk)))

    return pl.pallas_call(
        body,
        grid=grid,
        in_specs=[in_spec(j) for j in range(_PACK)],
        out_specs=pl.BlockSpec((blk, _PACK * D), lambda i: (i, 0)),
        out_shape=jax.ShapeDtypeStruct((_QROWS, _PACK * D), jnp.float32),
    )


def _make_sc_pooled_gather(B, L, D):
    b_per_w = B // _NW
    nsub = b_per_w // _SUB
    mesh = plsc.VectorSubcoreMesh(
        core_axis_name="c", subcore_axis_name="s",
        num_cores=_NC, num_subcores=_NS)

    @functools.partial(
        pl.kernel,
        out_type=tuple(jax.ShapeDtypeStruct((B, D), jnp.float32)
                       for _ in range(4)),
        mesh=mesh,
        scratch_types=[
            pltpu.VMEM((L, nsub, _SUB), jnp.int32),
            pltpu.VMEM((L, nsub, _SUB), jnp.int32),
            pltpu.VMEM((b_per_w, D), jnp.float32),
            pltpu.VMEM((b_per_w, D), jnp.float32),
            pltpu.VMEM((b_per_w, D), jnp.float32),
            pltpu.VMEM((b_per_w, D), jnp.float32),
            pltpu.SemaphoreType.DMA,
            pltpu.SemaphoreType.DMA,
        ],
        compiler_params=pltpu.CompilerParams(use_tc_tiling_on_sc=False,
                                             needs_layout_passes=False),
    )
    def sc_kernel(enc1t, enc2t, emb_hbm, e1e_out, e1o_out, e2e_out, e2o_out,
                  idx1_v, idx2_v, a1e_v, a1o_v, a2e_v, a2o_v, sem_e, sem_o):
        wid = lax.axis_index("s") * _NC + lax.axis_index("c")
        base = wid * b_per_w

        pltpu.sync_copy(enc1t.at[:, pl.ds(wid * nsub, nsub), :], idx1_v)
        pltpu.sync_copy(enc2t.at[:, pl.ds(wid * nsub, nsub), :], idx2_v)

        def remap(l):
            for idx_v in (idx1_v, idx2_v):
                for c in range(nsub):
                    for g in range(_SUB // 16):
                        v = idx_v[l, c, pl.ds(g * 16, 16)]
                        idx_v[l, c, pl.ds(g * 16, 16)] = (
                            (v & (_QROWS - 1)) * _PACK + (v >> 18))

        def remap_body(l, carry):
            remap(l)
            return carry

        lax.fori_loop(0, L, remap_body, 0)

        def fire(l, accs, sem, add):
            ds = []
            for idx_v, acc_v in ((idx1_v, accs[0]), (idx2_v, accs[1])):
                for c in range(nsub):
                    ds.append(pltpu.async_copy(
                        emb_hbm.at[idx_v.at[l, c]],
                        acc_v.at[pl.ds(c * _SUB, _SUB)],
                        sem, add=add))
            return ds

        def drain(accs, sem):
            for acc_v in accs:
                for c in range(nsub):
                    pltpu.make_async_copy(
                        emb_hbm.at[pl.ds(0, _SUB)],
                        acc_v.at[pl.ds(c * _SUB, _SUB)], sem).wait()

        even = (a1e_v, a2e_v)
        odd = (a1o_v, a2o_v)

        fire(0, even, sem_e, False)
        fire(1, odd, sem_o, False)

        def body(k, carry):
            drain(even, sem_e)
            fire(2 * k, even, sem_e, True)
            drain(odd, sem_o)
            fire(2 * k + 1, odd, sem_o, True)
            return carry

        lax.fori_loop(1, L // 2, body, 0)
        drain(even, sem_e)
        drain(odd, sem_o)

        pltpu.sync_copy(a1e_v, e1e_out.at[pl.ds(base, b_per_w)])
        pltpu.sync_copy(a1o_v, e1o_out.at[pl.ds(base, b_per_w)])
        pltpu.sync_copy(a2e_v, e2e_out.at[pl.ds(base, b_per_w)])
        pltpu.sync_copy(a2o_v, e2o_out.at[pl.ds(base, b_per_w)])

    return sc_kernel


def _make_tc_mlp(B, L, D, n_types, nhid, nclasses):
    blk = 512
    grid = (B // blk,)

    def body(e1e_ref, e1o_ref, e2e_ref, e2o_ref, c1_ref, c2_ref,
             w1_ref, b1_ref, w2_ref, b2_ref, out_ref):
        n1 = jnp.sum((c1_ref[...] != n_types).astype(jnp.float32), axis=0,
                     keepdims=True).T
        n2 = jnp.sum((c2_ref[...] != n_types).astype(jnp.float32), axis=0,
                     keepdims=True).T
        e1 = (e1e_ref[...] + e1o_ref[...]) / n1
        e2 = (e2e_ref[...] + e2o_ref[...]) / n2
        feat = jnp.concatenate([e1, e2, e1 * e2, jnp.abs(e1 - e2)], axis=1)
        h = lax.dot_general(feat, w1_ref[...], (((1,), (1,)), ((), ())),
                            preferred_element_type=jnp.float32) + b1_ref[...]
        h = jax.nn.sigmoid(h)
        out_ref[...] = lax.dot_general(h, w2_ref[...], (((1,), (1,)), ((), ())),
                                       preferred_element_type=jnp.float32
                                       ) + b2_ref[...]

    return pl.pallas_call(
        body,
        grid=grid,
        in_specs=[
            pl.BlockSpec((blk, D), lambda i: (i, 0)),
            pl.BlockSpec((blk, D), lambda i: (i, 0)),
            pl.BlockSpec((blk, D), lambda i: (i, 0)),
            pl.BlockSpec((blk, D), lambda i: (i, 0)),
            pl.BlockSpec((L, blk), lambda i: (0, i)),
            pl.BlockSpec((L, blk), lambda i: (0, i)),
            pl.BlockSpec((nhid, 4 * D), lambda i: (0, 0)),
            pl.BlockSpec((1, nhid), lambda i: (0, 0)),
            pl.BlockSpec((nclasses, nhid), lambda i: (0, 0)),
            pl.BlockSpec((1, nclasses), lambda i: (0, 0)),
        ],
        out_specs=pl.BlockSpec((blk, nclasses), lambda i: (i, 0)),
        out_shape=jax.ShapeDtypeStruct((B, nclasses), jnp.float32),
    )


def kernel(enc1, enc2, emb, W1, b1, W2, b2):
    B, L = enc1.shape
    N, D = emb.shape
    n_types = N - 1
    nhid = W1.shape[0]
    nclasses = W2.shape[0]

    enc1t = enc1.T.reshape(L, B // _SUB, _SUB)
    enc2t = enc2.T.reshape(L, B // _SUB, _SUB)
    embT = emb.T
    emb_flat = _make_tc_pack_table(N, D)(embT, embT, embT, embT)
    emb_sc = emb_flat.reshape(_ROWS_PAD, D)

    e1e, e1o, e2e, e2o = _make_sc_pooled_gather(B, L, D)(enc1t, enc2t, emb_sc)
    return _make_tc_mlp(B, L, D, n_types, nhid, nclasses)(
        e1e, e1o, e2e, e2o, enc1.T, enc2.T, W1, b1.reshape(1, nhid),
        W2, b2.reshape(1, nclasses))


# speedup vs baseline: 1.1668x; 1.0374x over previous
import functools

import jax
import jax.numpy as jnp
from jax import lax
from jax.experimental import pallas as pl
from jax.experimental.pallas import tpu as pltpu
from jax.experimental.pallas import tpu_sc as plsc

_NC = 2
_NS = 16
_NW = _NC * _NS
_SUB = 128
_PACK = 4
_QROWS = 262144
_ROWS_PAD = _PACK * _QROWS


def _make_tc_pack_table(N, D):
    blk = 8192
    nblk = _QROWS // blk
    grid = (nblk,)

    def body(t0_ref, t1_ref, t2_ref, t3_ref, out_ref):
        eye = jnp.eye(D, dtype=jnp.float32)
        for j, t_ref in enumerate((t0_ref, t1_ref, t2_ref, t3_ref)):
            out_ref[:, pl.ds(j * D, D)] = lax.dot_general(
                t_ref[...], eye, (((0,), (0,)), ((), ())),
                preferred_element_type=jnp.float32)

    max_blk = (N + blk - 1) // blk - 1

    def in_spec(j):
        return pl.BlockSpec(
            (D, blk), lambda i, j=j: (0, jnp.minimum(nblk * j + i, max_blk)))

    return pl.pallas_call(
        body,
        grid=grid,
        in_specs=[in_spec(j) for j in range(_PACK)],
        out_specs=pl.BlockSpec((blk, _PACK * D), lambda i: (i, 0)),
        out_shape=jax.ShapeDtypeStruct((_QROWS, _PACK * D), jnp.float32),
    )


def _make_sc_pooled_gather(B, L, D):
    b_per_w = B // _NW
    nsub = b_per_w // _SUB
    mesh = plsc.VectorSubcoreMesh(
        core_axis_name="c", subcore_axis_name="s",
        num_cores=_NC, num_subcores=_NS)

    @functools.partial(
        pl.kernel,
        out_type=tuple(jax.ShapeDtypeStruct((B, D), jnp.float32)
                       for _ in range(4)),
        mesh=mesh,
        scratch_types=[
            pltpu.VMEM((L, nsub, _SUB), jnp.int32),
            pltpu.VMEM((L, nsub, _SUB), jnp.int32),
            pltpu.VMEM((b_per_w, D), jnp.float32),
            pltpu.VMEM((b_per_w, D), jnp.float32),
            pltpu.VMEM((b_per_w, D), jnp.float32),
            pltpu.VMEM((b_per_w, D), jnp.float32),
            pltpu.SemaphoreType.DMA,
            pltpu.SemaphoreType.DMA,
        ],
        compiler_params=pltpu.CompilerParams(use_tc_tiling_on_sc=False,
                                             needs_layout_passes=False),
    )
    def sc_kernel(enc1t, enc2t, emb_hbm, e1e_out, e1o_out, e2e_out, e2o_out,
                  idx1_v, idx2_v, a1e_v, a1o_v, a2e_v, a2o_v, sem_e, sem_o):
        wid = lax.axis_index("s") * _NC + lax.axis_index("c")
        base = wid * b_per_w

        pltpu.sync_copy(enc1t.at[:, pl.ds(wid * nsub, nsub), :], idx1_v)
        pltpu.sync_copy(enc2t.at[:, pl.ds(wid * nsub, nsub), :], idx2_v)

        def remap(l):
            for idx_v in (idx1_v, idx2_v):
                for c in range(nsub):
                    for g in range(_SUB // 16):
                        v = idx_v[l, c, pl.ds(g * 16, 16)]
                        idx_v[l, c, pl.ds(g * 16, 16)] = (
                            (v & (_QROWS - 1)) * _PACK + (v >> 18))

        def remap_body(l, carry):
            remap(l)
            return carry

        lax.fori_loop(0, L, remap_body, 0)

        def fire(l, accs, sem, add):
            ds = []
            for idx_v, acc_v in ((idx1_v, accs[0]), (idx2_v, accs[1])):
                for c in range(nsub):
                    ds.append(pltpu.async_copy(
                        emb_hbm.at[idx_v.at[l, c]],
                        acc_v.at[pl.ds(c * _SUB, _SUB)],
                        sem, add=add))
            return ds

        def drain(accs, sem):
            for acc_v in accs:
                for c in range(nsub):
                    pltpu.make_async_copy(
                        emb_hbm.at[pl.ds(0, _SUB)],
                        acc_v.at[pl.ds(c * _SUB, _SUB)], sem).wait()

        even = (a1e_v, a2e_v)
        odd = (a1o_v, a2o_v)

        fire(0, even, sem_e, False)
        fire(1, odd, sem_o, False)

        def body(k, carry):
            drain(even, sem_e)
            fire(2 * k, even, sem_e, True)
            drain(odd, sem_o)
            fire(2 * k + 1, odd, sem_o, True)
            return carry

        lax.fori_loop(1, L // 2, body, 0)
        drain(even, sem_e)
        drain(odd, sem_o)

        pltpu.sync_copy(a1e_v, e1e_out.at[pl.ds(base, b_per_w)])
        pltpu.sync_copy(a1o_v, e1o_out.at[pl.ds(base, b_per_w)])
        pltpu.sync_copy(a2e_v, e2e_out.at[pl.ds(base, b_per_w)])
        pltpu.sync_copy(a2o_v, e2o_out.at[pl.ds(base, b_per_w)])

    return sc_kernel


def _make_tc_mlp(B, L, D, n_types, nhid, nclasses):
    blk = 2048
    grid = (B // blk,)

    def body(e1e_ref, e1o_ref, e2e_ref, e2o_ref, c1_ref, c2_ref,
             w1_ref, b1_ref, w2_ref, b2_ref, out_ref):
        n1 = jnp.sum((c1_ref[...] != n_types).astype(jnp.float32), axis=0,
                     keepdims=True).T
        n2 = jnp.sum((c2_ref[...] != n_types).astype(jnp.float32), axis=0,
                     keepdims=True).T
        e1 = (e1e_ref[...] + e1o_ref[...]) / n1
        e2 = (e2e_ref[...] + e2o_ref[...]) / n2
        feat = jnp.concatenate([e1, e2, e1 * e2, jnp.abs(e1 - e2)], axis=1)
        h = lax.dot_general(feat, w1_ref[...], (((1,), (1,)), ((), ())),
                            preferred_element_type=jnp.float32) + b1_ref[...]
        h = jax.nn.sigmoid(h)
        out_ref[...] = lax.dot_general(h, w2_ref[...], (((1,), (1,)), ((), ())),
                                       preferred_element_type=jnp.float32
                                       ) + b2_ref[...]

    return pl.pallas_call(
        body,
        grid=grid,
        in_specs=[
            pl.BlockSpec((blk, D), lambda i: (i, 0)),
            pl.BlockSpec((blk, D), lambda i: (i, 0)),
            pl.BlockSpec((blk, D), lambda i: (i, 0)),
            pl.BlockSpec((blk, D), lambda i: (i, 0)),
            pl.BlockSpec((L, blk), lambda i: (0, i)),
            pl.BlockSpec((L, blk), lambda i: (0, i)),
            pl.BlockSpec((nhid, 4 * D), lambda i: (0, 0)),
            pl.BlockSpec((1, nhid), lambda i: (0, 0)),
            pl.BlockSpec((nclasses, nhid), lambda i: (0, 0)),
            pl.BlockSpec((1, nclasses), lambda i: (0, 0)),
        ],
        out_specs=pl.BlockSpec((blk, nclasses), lambda i: (i, 0)),
        out_shape=jax.ShapeDtypeStruct((B, nclasses), jnp.float32),
    )


def kernel(enc1, enc2, emb, W1, b1, W2, b2):
    B, L = enc1.shape
    N, D = emb.shape
    n_types = N - 1
    nhid = W1.shape[0]
    nclasses = W2.shape[0]

    enc1t = enc1.T.reshape(L, B // _SUB, _SUB)
    enc2t = enc2.T.reshape(L, B // _SUB, _SUB)
    embT = emb.T
    emb_flat = _make_tc_pack_table(N, D)(embT, embT, embT, embT)
    emb_sc = emb_flat.reshape(_ROWS_PAD, D)

    e1e, e1o, e2e, e2o = _make_sc_pooled_gather(B, L, D)(enc1t, enc2t, emb_sc)
    return _make_tc_mlp(B, L, D, n_types, nhid, nclasses)(
        e1e, e1o, e2e, e2o, enc1.T, enc2.T, W1, b1.reshape(1, nhid),
        W2, b2.reshape(1, nclasses))
